# baseline (speedup 1.0000x reference)
import functools

import jax
import jax.numpy as jnp
from jax import lax
from jax.experimental import pallas as pl
from jax.experimental.pallas import tpu as pltpu
from jax.experimental.pallas import tpu_sc as plsc

NUM_EMBEDDINGS = 8192
EMBEDDING_DIM = 256
TOKEN_TILE = 512
K_BLOCK = 2048
N_CHUNKS = 1


def _argmin_body(z_ref, e_ref, z2h_ref, e2h_ref, out_ref):
    z = z_ref[...][0]
    z2h = z2h_ref[...]
    n_blocks = NUM_EMBEDDINGS // K_BLOCK
    m_run = None
    i_run = None
    for j in range(n_blocks):
        ej = e_ref[pl.ds(j * K_BLOCK, K_BLOCK), :]
        e2j = e2h_ref[:, pl.ds(j * K_BLOCK, K_BLOCK)]
        mm = lax.dot_general(
            z, ej, (((0,), (1,)), ((), ())),
            preferred_element_type=jnp.float32,
        )
        d = (z2h + e2j) - mm
        bm = jnp.min(d, axis=1, keepdims=True)
        iota = lax.broadcasted_iota(
            jnp.int32, (1, K_BLOCK), 1).astype(jnp.float32) + (j * K_BLOCK)
        bi = jnp.min(
            jnp.where(d == bm, iota, jnp.float32(NUM_EMBEDDINGS)),
            axis=1, keepdims=True)
        if m_run is None:
            m_run, i_run = bm, bi
        else:
            upd = bm < m_run
            i_run = jnp.where(upd, bi, i_run)
            m_run = jnp.minimum(bm, m_run)
    out_ref[...] = i_run.astype(jnp.int32)


@functools.partial(jax.jit, static_argnames=())
def _encode(z_cm, embedding_weight, z2h, e2h):
    B_, C_, HW_ = z_cm.shape
    n_tok = B_ * HW_
    tiles_per_b = HW_ // TOKEN_TILE
    grid = (n_tok // TOKEN_TILE,)
    return pl.pallas_call(
        _argmin_body,
        grid=grid,
        in_specs=[
            pl.BlockSpec((1, C_, TOKEN_TILE),
                         lambda i: (i // tiles_per_b, 0, i % tiles_per_b)),
            pl.BlockSpec((NUM_EMBEDDINGS, EMBEDDING_DIM), lambda i: (0, 0)),
            pl.BlockSpec((TOKEN_TILE, 1), lambda i: (i, 0)),
            pl.BlockSpec((1, NUM_EMBEDDINGS), lambda i: (0, 0)),
        ],
        out_specs=pl.BlockSpec((TOKEN_TILE, 1), lambda i: (i, 0)),
        out_shape=jax.ShapeDtypeStruct((n_tok, 1), jnp.int32),
    )(z_cm, embedding_weight, z2h, e2h)


def _make_gather(n_rows):
    info = plsc.get_sparse_core_info()
    n_workers = info.num_cores * info.num_subcores
    rows_per_w = n_rows // n_workers
    n_sub = (rows_per_w + 127) // 128
    sub = rows_per_w // n_sub

    @functools.partial(
        pl.kernel,
        out_type=jax.ShapeDtypeStruct((n_rows, EMBEDDING_DIM), jnp.float32),
        mesh=plsc.VectorSubcoreMesh(core_axis_name="c", subcore_axis_name="s"),
        scratch_types=[
            pltpu.VMEM((rows_per_w,), jnp.int32),
            pltpu.VMEM((rows_per_w, EMBEDDING_DIM), jnp.float32),
            pltpu.SemaphoreType.DMA,
        ],
    )
    def gather(idx_hbm, table_hbm, out_hbm, idx_v, rows_v, sem):
        wid = lax.axis_index("s") * info.num_cores + lax.axis_index("c")
        base = wid * rows_per_w
        pltpu.sync_copy(idx_hbm.at[pl.ds(base, rows_per_w)], idx_v)
        copies = [
            pltpu.async_copy(
                table_hbm.at[idx_v.at[pl.ds(j * sub, sub)]],
                rows_v.at[pl.ds(j * sub, sub)], sem)
            for j in range(n_sub)
        ]
        for c in copies:
            c.wait()
        pltpu.sync_copy(rows_v, out_hbm.at[pl.ds(base, rows_per_w)])

    return gather


def kernel(z_e, embedding_weight):
    B, C, H, W = z_e.shape
    n_tok = B * H * W
    z_cm = z_e.reshape(B, C, H * W)
    z_flat = jnp.transpose(z_cm, (0, 2, 1))
    z2 = jnp.sum(z_flat ** 2, axis=2, keepdims=True)
    e2 = jnp.sum(embedding_weight ** 2, axis=1)
    z2h = (z2 * 0.5).reshape(n_tok, 1)
    e2h = (e2 * 0.5).reshape(1, NUM_EMBEDDINGS)

    gather = _make_gather(n_tok)
    idx = _encode(z_cm, embedding_weight, z2h, e2h)
    quantized = gather(idx.reshape(n_tok), embedding_weight)
    encoding_indices = idx.reshape(B, H * W)
    quantized = quantized.reshape(B, H * W, C)
    quantized = jnp.transpose(quantized, (0, 2, 1)).reshape(B, C, H, W)
    return (quantized, encoding_indices)


# speedup vs baseline: 1.1232x; 1.1232x over previous
import functools

import jax
import jax.numpy as jnp
from jax import lax
from jax.experimental import pallas as pl
from jax.experimental.pallas import tpu as pltpu
from jax.experimental.pallas import tpu_sc as plsc

NUM_EMBEDDINGS = 8192
EMBEDDING_DIM = 256
TOKEN_TILE = 512
K_BLOCK = 1024


def _argmin_body(z_ref, e_ref, z2h_ref, e2h_ref, out_ref):
    z = z_ref[...]
    z2h = z2h_ref[...]
    n_blocks = NUM_EMBEDDINGS // K_BLOCK
    m_run = None
    i_run = None
    for j in range(n_blocks):
        ej = e_ref[pl.ds(j * K_BLOCK, K_BLOCK), :]
        e2j = e2h_ref[:, pl.ds(j * K_BLOCK, K_BLOCK)]
        mm = lax.dot_general(
            z, ej, (((1,), (1,)), ((), ())),
            preferred_element_type=jnp.float32,
        )
        d = (z2h + e2j) - mm
        bm = jnp.min(d, axis=1, keepdims=True)
        iota = lax.broadcasted_iota(
            jnp.int32, (1, K_BLOCK), 1).astype(jnp.float32) + (j * K_BLOCK)
        bi = jnp.min(
            jnp.where(d == bm, iota, jnp.float32(NUM_EMBEDDINGS)),
            axis=1, keepdims=True)
        if m_run is None:
            m_run, i_run = bm, bi
        else:
            upd = bm < m_run
            i_run = jnp.where(upd, bi, i_run)
            m_run = jnp.minimum(bm, m_run)
    out_ref[...] = i_run.astype(jnp.int32)


@functools.partial(jax.jit, static_argnames=())
def _encode(z_flat, embedding_weight, z2h, e2h):
    n_tok = z_flat.shape[0]
    grid = (n_tok // TOKEN_TILE,)
    return pl.pallas_call(
        _argmin_body,
        grid=grid,
        in_specs=[
            pl.BlockSpec((TOKEN_TILE, EMBEDDING_DIM), lambda i: (i, 0)),
            pl.BlockSpec((NUM_EMBEDDINGS, EMBEDDING_DIM), lambda i: (0, 0)),
            pl.BlockSpec((TOKEN_TILE, 1), lambda i: (i, 0)),
            pl.BlockSpec((1, NUM_EMBEDDINGS), lambda i: (0, 0)),
        ],
        out_specs=pl.BlockSpec((TOKEN_TILE, 1), lambda i: (i, 0)),
        out_shape=jax.ShapeDtypeStruct((n_tok, 1), jnp.int32),
    )(z_flat, embedding_weight, z2h, e2h)


def _make_gather(n_rows):
    info = plsc.get_sparse_core_info()
    n_workers = info.num_cores * info.num_subcores
    rows_per_w = n_rows // n_workers
    n_sub = (rows_per_w + 127) // 128
    sub = rows_per_w // n_sub

    @functools.partial(
        pl.kernel,
        out_type=jax.ShapeDtypeStruct((n_rows, EMBEDDING_DIM), jnp.float32),
        mesh=plsc.VectorSubcoreMesh(core_axis_name="c", subcore_axis_name="s"),
        scratch_types=[
            pltpu.VMEM((rows_per_w,), jnp.int32),
            pltpu.VMEM((rows_per_w, EMBEDDING_DIM), jnp.float32),
            pltpu.SemaphoreType.DMA,
        ],
    )
    def gather(idx_hbm, table_hbm, out_hbm, idx_v, rows_v, sem):
        wid = lax.axis_index("s") * info.num_cores + lax.axis_index("c")
        base = wid * rows_per_w
        pltpu.sync_copy(idx_hbm.at[pl.ds(base, rows_per_w)], idx_v)
        copies = [
            pltpu.async_copy(
                table_hbm.at[idx_v.at[pl.ds(j * sub, sub)]],
                rows_v.at[pl.ds(j * sub, sub)], sem)
            for j in range(n_sub)
        ]
        for c in copies:
            c.wait()
        pltpu.sync_copy(rows_v, out_hbm.at[pl.ds(base, rows_per_w)])

    return gather


def kernel(z_e, embedding_weight):
    B, C, H, W = z_e.shape
    n_tok = B * H * W
    z_flat = jnp.transpose(z_e.reshape(B, C, H * W), (0, 2, 1))
    z2 = jnp.sum(z_flat ** 2, axis=2, keepdims=True)
    e2 = jnp.sum(embedding_weight ** 2, axis=1)
    z2h = (z2 * 0.5).reshape(n_tok, 1)
    e2h = (e2 * 0.5).reshape(1, NUM_EMBEDDINGS)

    gather = _make_gather(n_tok)
    idx = _encode(z_flat.reshape(n_tok, C), embedding_weight, z2h, e2h)
    quantized = gather(idx.reshape(n_tok), embedding_weight)
    encoding_indices = idx.reshape(B, H * W)
    quantized = quantized.reshape(B, H * W, C)
    quantized = jnp.transpose(quantized, (0, 2, 1)).reshape(B, C, H, W)
    return (quantized, encoding_indices)
